# baseline (speedup 1.0000x reference)
import functools

import jax
import jax.numpy as jnp
from jax import lax
from jax.experimental import pallas as pl
from jax.experimental.pallas import tpu as pltpu
from jax.experimental.pallas import tpu_sc as plsc

_B = 16384
_NC = 2
_NS = 16
_NW = _NC * _NS
_BPW = _B // _NW
_CHUNK = 128
_NCHUNK = 4 * _BPW // _CHUNK
_SLOTS = 12
_LEAD = 6


def _sc_body(coords_hbm, table_hbm, out_hbm, coords_v, idx_v, table_sh,
             bufs, gsems, wsems, stage_sem):
    cid = lax.axis_index("c")
    sid = lax.axis_index("s")
    wid = sid * _NC + cid
    base = wid * _BPW
    out_base = wid * (4 * _BPW)

    @pl.when(sid == 0)
    def _():
        pltpu.async_copy(table_hbm, table_sh, stage_sem)

    pltpu.sync_copy(coords_hbm.at[pl.ds(3 * base, 3 * _BPW)], coords_v)

    lane = lax.iota(jnp.int32, 16)

    def build_chunk(ck):
        for i in (2 * ck, 2 * ck + 1):
            b16 = i * 16 + lane
            f = b16 * 3
            r = plsc.load_gather(coords_v, [f])
            c = plsc.load_gather(coords_v, [f + 1])
            z = plsc.load_gather(coords_v, [f + 2])
            p = b16 * 4
            for off, val in ((0, 2 * r), (1, 2 * c), (2, 2 * z),
                             (3, 2 * z + 1)):
                plsc.store_scatter(idx_v, [p + off], val)

    @pl.when(sid == 0)
    def _():
        pltpu.make_async_copy(table_hbm, table_sh, stage_sem).wait()

    plsc.subcore_barrier()

    def gather(c):
        s = c % _SLOTS
        return pltpu.async_copy(
            table_sh.at[idx_v.at[pl.ds(c * _CHUNK, _CHUNK)]],
            bufs[s], gsems[s])

    def write(c):
        s = c % _SLOTS
        return pltpu.async_copy(
            bufs[s], out_hbm.at[pl.ds(out_base + c * _CHUNK, _CHUNK)],
            wsems[s])

    gcp = {}
    wcp = {}
    for c in range(_LEAD):
        build_chunk(c)
        gcp[c] = gather(c)
    for c in range(_NCHUNK):
        g = c + _LEAD
        if g < _NCHUNK:
            build_chunk(g)
            if g - _SLOTS >= 0:
                wcp[g - _SLOTS].wait()
            gcp[g] = gather(g)
        gcp[c].wait()
        wcp[c] = write(c)
    for c in range(_NCHUNK - _SLOTS, _NCHUNK):
        wcp[c].wait()


_sc_call = functools.partial(
    pl.kernel,
    out_type=jax.ShapeDtypeStruct((4 * _B, 64), jnp.float32),
    mesh=plsc.VectorSubcoreMesh(core_axis_name="c", subcore_axis_name="s"),
    compiler_params=pltpu.CompilerParams(
        needs_layout_passes=False, use_tc_tiling_on_sc=False),
    scratch_types=[
        pltpu.VMEM((3 * _BPW,), jnp.int32),
        pltpu.VMEM((_NCHUNK * _CHUNK,), jnp.int32),
        pltpu.VMEM_SHARED((2 * 1000, 64), jnp.float32),
        [pltpu.VMEM((_CHUNK, 64), jnp.float32) for _ in range(_SLOTS)],
        [pltpu.SemaphoreType.DMA for _ in range(_SLOTS)],
        [pltpu.SemaphoreType.DMA for _ in range(_SLOTS)],
        pltpu.SemaphoreType.DMA,
    ],
)(_sc_body)


@jax.jit
def kernel(spatial_coord, pos_enc):
    table = pos_enc.reshape(2 * 1000, 64)
    out = _sc_call(spatial_coord.reshape(3 * _B), table)
    return out.reshape(_B, 2, 128)


# speedup vs baseline: 1.0088x; 1.0088x over previous
import functools

import jax
import jax.numpy as jnp
from jax import lax
from jax.experimental import pallas as pl
from jax.experimental.pallas import tpu as pltpu
from jax.experimental.pallas import tpu_sc as plsc

_B = 16384
_NC = 2
_NS = 16
_NW = _NC * _NS
_BPW = _B // _NW
_CHUNK = 128
_NCHUNK = 4 * _BPW // _CHUNK
_SLOTS = 8
_LEAD = 4


def _sc_body(coords_hbm, table_hbm, out_hbm, coords_v, idx_v, table_sh,
             bufs, gsems, wsems, stage_sem):
    cid = lax.axis_index("c")
    sid = lax.axis_index("s")
    wid = sid * _NC + cid
    base = wid * _BPW
    out_base = wid * (4 * _BPW)

    @pl.when(sid == 0)
    def _():
        pltpu.async_copy(table_hbm, table_sh, stage_sem)

    pltpu.sync_copy(coords_hbm.at[pl.ds(3 * base, 3 * _BPW)], coords_v)

    lane = lax.iota(jnp.int32, 16)

    def build(i, carry):
        b16 = i * 16 + lane
        f = b16 * 3
        r = plsc.load_gather(coords_v, [f])
        c = plsc.load_gather(coords_v, [f + 1])
        z = plsc.load_gather(coords_v, [f + 2])
        p = b16 * 4
        for off, val in ((0, 2 * r), (1, 2 * c), (2, 2 * z), (3, 2 * z + 1)):
            plsc.store_scatter(idx_v, [p + off], val)
        return carry

    lax.fori_loop(0, _BPW // 16, build, 0)

    @pl.when(sid == 0)
    def _():
        pltpu.make_async_copy(table_hbm, table_sh, stage_sem).wait()

    plsc.subcore_barrier()

    def gather(c):
        s = c % _SLOTS
        return pltpu.async_copy(
            table_sh.at[idx_v.at[pl.ds(c * _CHUNK, _CHUNK)]],
            bufs[s], gsems[s])

    def write(c):
        s = c % _SLOTS
        return pltpu.async_copy(
            bufs[s], out_hbm.at[pl.ds(out_base + c * _CHUNK, _CHUNK)],
            wsems[s])

    gcp = {}
    wcp = {}
    for c in range(_LEAD):
        gcp[c] = gather(c)
    for c in range(_NCHUNK):
        g = c + _LEAD
        if g < _NCHUNK:
            if g - _SLOTS >= 0:
                wcp[g - _SLOTS].wait()
            gcp[g] = gather(g)
        gcp[c].wait()
        wcp[c] = write(c)
    for c in range(_NCHUNK - _SLOTS, _NCHUNK):
        wcp[c].wait()


_sc_call = functools.partial(
    pl.kernel,
    out_type=jax.ShapeDtypeStruct((4 * _B, 64), jnp.float32),
    mesh=plsc.VectorSubcoreMesh(core_axis_name="c", subcore_axis_name="s"),
    compiler_params=pltpu.CompilerParams(
        needs_layout_passes=False, use_tc_tiling_on_sc=False),
    scratch_types=[
        pltpu.VMEM((3 * _BPW,), jnp.int32),
        pltpu.VMEM((_NCHUNK * _CHUNK,), jnp.int32),
        pltpu.VMEM_SHARED((2 * 1000, 64), jnp.float32),
        [pltpu.VMEM((_CHUNK, 64), jnp.float32) for _ in range(_SLOTS)],
        [pltpu.SemaphoreType.DMA for _ in range(_SLOTS)],
        [pltpu.SemaphoreType.DMA for _ in range(_SLOTS)],
        pltpu.SemaphoreType.DMA,
    ],
)(_sc_body)


@jax.jit
def kernel(spatial_coord, pos_enc):
    table = pos_enc.reshape(2 * 1000, 64)
    out = _sc_call(spatial_coord.reshape(3 * _B), table)
    return out.reshape(_B, 2, 128)


# speedup vs baseline: 1.0107x; 1.0018x over previous
import functools

import jax
import jax.numpy as jnp
from jax import lax
from jax.experimental import pallas as pl
from jax.experimental.pallas import tpu as pltpu
from jax.experimental.pallas import tpu_sc as plsc

_B = 16384
_NC = 2
_NS = 16
_NW = _NC * _NS
_BPW = _B // _NW
_CHUNK = 128
_NCHUNK = 4 * _BPW // _CHUNK
_SLOTS = 8
_LEAD = 4


def _sc_body(coords_hbm, table_hbm, out_hbm, coords_v, idx_v, table_sh,
             bufs, gsems, wsems, stage_sem):
    cid = lax.axis_index("c")
    sid = lax.axis_index("s")
    wid = sid * _NC + cid
    base = wid * _BPW
    out_base = wid * (4 * _BPW)

    @pl.when(sid == 0)
    def _():
        pltpu.async_copy(table_hbm, table_sh, stage_sem)

    pltpu.sync_copy(coords_hbm.at[pl.ds(3 * base, 3 * _BPW)], coords_v)

    lane = lax.iota(jnp.int32, 16)

    def build(i, carry):
        b16 = i * 16 + lane
        f = b16 * 3
        r = plsc.load_gather(coords_v, [f])
        c = plsc.load_gather(coords_v, [f + 1])
        z = plsc.load_gather(coords_v, [f + 2])
        p = b16 * 4
        for off, val in ((0, 2 * r), (1, 2 * c), (2, 2 * z), (3, 2 * z + 1)):
            plsc.store_scatter(idx_v, [p + off], val)
        return carry

    lax.fori_loop(0, _BPW // 16, build, 0)

    @pl.when(sid == 0)
    def _():
        pltpu.make_async_copy(table_hbm, table_sh, stage_sem).wait()

    plsc.subcore_barrier()

    def gather(c):
        s = c % _SLOTS
        return pltpu.async_copy(
            table_sh.at[idx_v.at[pl.ds(c * _CHUNK, _CHUNK)]],
            bufs[s], gsems[s])

    def write(c):
        s = c % _SLOTS
        return pltpu.async_copy(
            bufs[s], out_hbm.at[pl.ds(out_base + c * _CHUNK, _CHUNK)],
            wsems[s])

    gcp = {}
    wcp = {}
    for c in range(_LEAD):
        gcp[c] = gather(c)
    for c in range(_NCHUNK):
        g = c + _LEAD
        if g < _NCHUNK:
            if g - _SLOTS >= 0:
                wcp[g - _SLOTS].wait()
            gcp[g] = gather(g)
        gcp[c].wait()
        wcp[c] = write(c)
    for c in range(_NCHUNK - _SLOTS, _NCHUNK):
        wcp[c].wait()


_sc_call = functools.partial(
    pl.kernel,
    out_type=jax.ShapeDtypeStruct((4 * _B, 64), jnp.float32),
    mesh=plsc.VectorSubcoreMesh(core_axis_name="c", subcore_axis_name="s"),
    compiler_params=pltpu.CompilerParams(
        needs_layout_passes=False, use_tc_tiling_on_sc=False,
        disable_bounds_checks=True, disable_semaphore_checks=True,
        skip_device_barrier=True),
    scratch_types=[
        pltpu.VMEM((3 * _BPW,), jnp.int32),
        pltpu.VMEM((_NCHUNK * _CHUNK,), jnp.int32),
        pltpu.VMEM_SHARED((2 * 1000, 64), jnp.float32),
        [pltpu.VMEM((_CHUNK, 64), jnp.float32) for _ in range(_SLOTS)],
        [pltpu.SemaphoreType.DMA for _ in range(_SLOTS)],
        [pltpu.SemaphoreType.DMA for _ in range(_SLOTS)],
        pltpu.SemaphoreType.DMA,
    ],
)(_sc_body)


@jax.jit
def kernel(spatial_coord, pos_enc):
    table = pos_enc.reshape(2 * 1000, 64)
    out = _sc_call(spatial_coord.reshape(3 * _B), table)
    return out.reshape(_B, 2, 128)
